# baseline (speedup 1.0000x reference)
import functools

import jax
import jax.numpy as jnp
from jax import lax
from jax.experimental import pallas as pl
from jax.experimental.pallas import tpu as pltpu
from jax.experimental.pallas import tpu_sc as plsc

N_TOK = 16384
D_EMB = 2048
N_EXPERTS = 16
TOP_K = 2

BLK = 512
N_BUF = 4
SPLITS = 2
SUB = BLK // SPLITS
NSTEPS = N_TOK // BLK

NC = 2
NS = 16
NW = NC * NS
C_TOK = N_TOK // NW


def _h_copy(h_hbm, hbuf, sems, chunk, slot, s):
    return pltpu.make_async_copy(
        h_hbm.at[pl.ds(chunk * BLK + s * SUB, SUB), :],
        hbuf.at[slot, pl.ds(s * SUB, SUB), :],
        sems.at[slot, s])


def _logits_kernel(h_hbm, md_ref, w1_ref, b1_ref, w2_ref, b2_ref,
                   wg_ref, bg_ref, logits_ref, hbuf, sems):
    i = pl.program_id(0)
    slot = jax.lax.rem(i, N_BUF)

    @pl.when(i == 0)
    def _prologue():
        for c in range(min(N_BUF - 1, NSTEPS)):
            for s in range(SPLITS):
                _h_copy(h_hbm, hbuf, sems, c, c % N_BUF, s).start()

    @pl.when(i + N_BUF - 1 < NSTEPS)
    def _prefetch():
        nxt = i + N_BUF - 1
        nslot = jax.lax.rem(nxt, N_BUF)
        for s in range(SPLITS):
            _h_copy(h_hbm, hbuf, sems, nxt, nslot, s).start()

    for s in range(SPLITS):
        _h_copy(h_hbm, hbuf, sems, i, slot, s).wait()
    hb = hbuf[slot]

    md = md_ref[...]
    g = jnp.dot(md, w1_ref[...], preferred_element_type=jnp.float32)
    g = g + b1_ref[...]
    g = 0.5 * g * (1.0 + jax.lax.erf(g * 0.7071067811865476))
    m_emb = jnp.dot(g, w2_ref[...], preferred_element_type=jnp.float32)
    m_emb = m_emb + b2_ref[...]

    logits = jnp.dot(hb, wg_ref[:D_EMB, :], preferred_element_type=jnp.float32)
    logits = logits + jnp.dot(m_emb, wg_ref[D_EMB:, :],
                              preferred_element_type=jnp.float32)
    logits_ref[...] = logits + bg_ref[...]


def _route_kernel(logits_hbm, gw_hbm, idx_hbm, chunk_v, gw_v, idx_v):
    wid = lax.axis_index("s") * NC + lax.axis_index("c")
    base = wid * C_TOK
    pltpu.sync_copy(logits_hbm.at[pl.ds(base * N_EXPERTS, C_TOK * N_EXPERTS)],
                    chunk_v)
    lanes = lax.iota(jnp.int32, N_EXPERTS)
    neg_inf = jnp.float32(-jnp.inf)

    def body(t, carry):
        row = chunk_v[pl.ds(t * N_EXPERTS, N_EXPERTS)]
        v1 = jnp.max(row)
        is1 = row == v1
        first1 = jnp.logical_and(is1, jnp.cumsum(is1.astype(jnp.int32)) == 1)
        masked = jnp.where(first1, neg_inf, row)
        v2 = jnp.max(masked)
        is2 = masked == v2
        first2 = jnp.logical_and(is2, jnp.cumsum(is2.astype(jnp.int32)) == 1)
        keep = jnp.logical_or(first1, first2)
        ex = jnp.where(keep, jnp.exp(row - v1), jnp.float32(0.0))
        gw_v[pl.ds(t * N_EXPERTS, N_EXPERTS)] = ex / jnp.sum(ex)
        plsc.store_compressed(idx_v.at[pl.ds(2 * t, N_EXPERTS)], lanes, mask=first1)
        plsc.store_compressed(idx_v.at[pl.ds(2 * t + 1, N_EXPERTS)], lanes,
                              mask=first2)
        return carry

    lax.fori_loop(0, C_TOK, body, 0)
    pltpu.sync_copy(gw_v, gw_hbm.at[pl.ds(base * N_EXPERTS,
                                          C_TOK * N_EXPERTS)])
    pltpu.sync_copy(idx_v.at[pl.ds(0, C_TOK * TOP_K)],
                    idx_hbm.at[pl.ds(base * TOP_K, C_TOK * TOP_K)])


@functools.partial(jax.jit, static_argnames=())
def kernel(h, metadata, W1, b1, W2, b2, Wg, bg, mu):
    n_tok = h.shape[0]
    grid = (n_tok // BLK,)
    full = lambda shape: pl.BlockSpec(shape, lambda i: (0,) * len(shape))

    logits = pl.pallas_call(
        _logits_kernel,
        grid=grid,
        in_specs=[
            pl.BlockSpec(memory_space=pltpu.MemorySpace.HBM),
            pl.BlockSpec((BLK, 2), lambda i: (i, 0)),
            full((2, 16)),
            full((1, 16)),
            full((16, 8)),
            full((1, 8)),
            full((D_EMB + 8, N_EXPERTS)),
            full((1, N_EXPERTS)),
        ],
        out_specs=pl.BlockSpec((BLK, N_EXPERTS), lambda i: (i, 0)),
        out_shape=jax.ShapeDtypeStruct((n_tok, N_EXPERTS), jnp.float32),
        scratch_shapes=[
            pltpu.VMEM((N_BUF, BLK, D_EMB), jnp.float32),
            pltpu.SemaphoreType.DMA((N_BUF, SPLITS)),
        ],
        compiler_params=pltpu.CompilerParams(
            dimension_semantics=("arbitrary",),
        ),
    )(h, metadata, W1, b1.reshape(1, -1), W2, b2.reshape(1, -1),
      Wg, bg.reshape(1, -1))

    route = pl.kernel(
        _route_kernel,
        out_type=[
            jax.ShapeDtypeStruct((n_tok * N_EXPERTS,), jnp.float32),
            jax.ShapeDtypeStruct((n_tok * TOP_K,), jnp.int32),
        ],
        mesh=plsc.VectorSubcoreMesh(core_axis_name="c", subcore_axis_name="s"),
        compiler_params=pltpu.CompilerParams(needs_layout_passes=False),
        scratch_types=[
            pltpu.VMEM((C_TOK * N_EXPERTS,), jnp.float32),
            pltpu.VMEM((C_TOK * N_EXPERTS,), jnp.float32),
            pltpu.VMEM((C_TOK * TOP_K + N_EXPERTS,), jnp.int32),
        ],
    )
    gw_flat, idx_flat = route(logits.reshape(-1))
    return (gw_flat.reshape(n_tok, N_EXPERTS),
            idx_flat.reshape(n_tok, TOP_K).astype(jnp.int32), mu)
